# baseline (speedup 1.0000x reference)
import functools

import jax
import jax.numpy as jnp
from jax import lax
from jax.experimental import pallas as pl
from jax.experimental.pallas import tpu as pltpu
from jax.experimental.pallas import tpu_sc as plsc

B = 128
H = 8
V = 4096
NW = 32
R_SC = 64
RPW = R_SC // NW
LANE = 16
ACT_N = RPW * H
U = 8
LN2 = 0.6931471805599453


def _lane_perm(v, idx):
    return lax.gather(
        v, idx[:, None],
        dimension_numbers=lax.GatherDimensionNumbers(
            offset_dims=(), collapsed_slice_dims=(0,), start_index_map=(0,)),
        slice_sizes=(1,),
        mode=lax.GatherScatterMode.PROMISE_IN_BOUNDS)


def _tree(vs, op):
    vs = list(vs)
    while len(vs) > 1:
        vs = [op(vs[i], vs[i + 1]) for i in range(0, len(vs) - 1, 2)] + (
            [vs[-1]] if len(vs) % 2 else [])
    return vs[0]


def _allreduce(v, lanes, op, steps=(8, 4, 2, 1)):
    for k in steps:
        v = op(v, _lane_perm(v, lanes ^ k))
    return v


def _ln(s, lanes):
    bits = lax.bitcast_convert_type(s, jnp.int32)
    e = (bits >> 23) - 127
    f = lax.bitcast_convert_type((bits & 0x7FFFFF) | 0x3F800000,
                                 jnp.float32)
    t = f - 1.0
    z = t / (t + 2.0)
    z2 = z * z
    y = e.astype(jnp.float32) * LN2 + z * (2.0 + z2 * (2.0 / 3.0 + z2 * 0.4))
    for _ in range(2):
        y = y + (s * jnp.exp(-y) - 1.0)
    return y


FU = 4
NJ = V // (FU * LANE)


def _row_stats(buf, av_g, parity, lanes, carry):
    m_vec, s_vec, u_vec, xa_vec = carry
    base_lane = parity * H
    zeros = jnp.zeros((LANE,), jnp.float32)
    zerosU = tuple(zeros for _ in range(FU))
    neginfU = tuple(jnp.full((LANE,), -jnp.inf, jnp.float32)
                    for _ in range(FU))

    def at_of(h):
        return _lane_perm(av_g, jnp.full((LANE,), base_lane + h, jnp.int32))

    at0 = at_of(0)

    def p1_body(jj, st):
        maccs, xacc, cols = st
        base = jj * (FU * LANE)
        maccs = list(maccs)
        for k in range(FU):
            v = buf[pl.ds(base + k * LANE, LANE)]
            xacc = jnp.where(cols + k * LANE == at0, v, xacc)
            maccs[k] = jnp.maximum(maccs[k], v)
        return tuple(maccs), xacc, cols + FU * LANE

    maccs, xacc, _ = lax.fori_loop(0, NJ, p1_body, (neginfU, zeros, lanes))
    mt = _allreduce(_tree(maccs, jnp.maximum), lanes, jnp.maximum)
    xat = _lane_perm(xacc, at0 & (LANE - 1))

    def head_body(h, c):
        m_vec, s_vec, u_vec, xa_vec, mt, xat = c
        at = at_of(h)

        def fused_body(jj, st):
            maccs, xacc, cols, ss, uu = st
            b1 = h * V + jj * (FU * LANE)
            b2 = b1 - V
            maccs, ss, uu = list(maccs), list(ss), list(uu)
            for k in range(FU):
                v2 = buf[pl.ds(b2 + k * LANE, LANE)]
                d = v2 - mt
                ex = jnp.exp(d)
                ss[k] = ss[k] + ex
                uu[k] = uu[k] + d * ex
                v1 = buf[pl.ds(b1 + k * LANE, LANE)]
                xacc = jnp.where(cols + k * LANE == at, v1, xacc)
                maccs[k] = jnp.maximum(maccs[k], v1)
            return tuple(maccs), xacc, cols + FU * LANE, tuple(ss), tuple(uu)

        maccs, xacc, _, ss, uu = lax.fori_loop(
            0, NJ, fused_body, (neginfU, zeros, lanes, zerosU, zerosU))
        st = _allreduce(_tree(ss, jnp.add), lanes, jnp.add)
        ut = _allreduce(_tree(uu, jnp.add), lanes, jnp.add)
        sel = lanes == (base_lane + h - 1)
        c_out = (jnp.where(sel, mt, m_vec),
                 jnp.where(sel, st, s_vec),
                 jnp.where(sel, ut, u_vec),
                 jnp.where(sel, xat, xa_vec),
                 _allreduce(_tree(maccs, jnp.maximum), lanes, jnp.maximum),
                 _lane_perm(xacc, at & (LANE - 1)))
        return c_out

    m_vec, s_vec, u_vec, xa_vec, mt, xat = lax.fori_loop(
        1, H, head_body, (m_vec, s_vec, u_vec, xa_vec, mt, xat))

    def p2_body(jj, st):
        ss, uu = st
        base = (H - 1) * V + jj * (FU * LANE)
        ss, uu = list(ss), list(uu)
        for k in range(FU):
            v = buf[pl.ds(base + k * LANE, LANE)]
            d = v - mt
            ex = jnp.exp(d)
            ss[k] = ss[k] + ex
            uu[k] = uu[k] + d * ex
        return tuple(ss), tuple(uu)

    ss, uu = lax.fori_loop(0, NJ, p2_body, (zerosU, zerosU))
    st = _allreduce(_tree(ss, jnp.add), lanes, jnp.add)
    ut = _allreduce(_tree(uu, jnp.add), lanes, jnp.add)
    sel = lanes == (base_lane + H - 1)
    return (jnp.where(sel, mt, m_vec),
            jnp.where(sel, st, s_vec),
            jnp.where(sel, ut, u_vec),
            jnp.where(sel, xat, xa_vec))


def _sc_kernel(policy_hbm, actions_hbm, out_hbm, buf0, buf1, act_v,
               st_lp, st_ent, sem0, sem1):
    wid = lax.axis_index("s") * 2 + lax.axis_index("c")
    r0 = wid * RPW
    lanes = lax.iota(jnp.int32, LANE)

    pltpu.sync_copy(actions_hbm.at[pl.ds(wid * ACT_N, ACT_N)],
                    act_v.at[pl.ds(0, ACT_N)])

    bufs = (buf0, buf1)
    sems = (sem0, sem1)
    copies = [None] * RPW
    for i in range(min(2, RPW)):
        copies[i] = pltpu.async_copy(
            policy_hbm.at[r0 + i], bufs[i % 2], sems[i % 2])

    stage_lp = jnp.zeros((LANE,), jnp.float32)
    stage_ent = jnp.zeros((LANE,), jnp.float32)
    row_pick = (lanes & 1) * 8
    zeros = jnp.zeros((LANE,), jnp.float32)

    carry = (zeros, zeros, zeros, zeros)
    for i in range(RPW):
        g, parity = i // 2, i % 2
        if parity == 0:
            carry = (zeros, zeros, zeros, zeros)
        av_g = act_v[pl.ds(min(g * LANE, max(0, ACT_N - LANE)), LANE)]
        copies[i].wait()
        carry = _row_stats(bufs[i % 2], av_g, parity, lanes, carry)
        if i + 2 < RPW:
            copies[i + 2] = pltpu.async_copy(
                policy_hbm.at[r0 + i + 2], bufs[i % 2], sems[i % 2])
        if parity == 1 or i == RPW - 1:
            m_vec, s_vec, u_vec, xa_vec = carry
            ln_s = _ln(s_vec, lanes)
            lp = xa_vec - m_vec - ln_s
            ent = ln_s - u_vec / s_vec
            lp = _allreduce(lp, lanes, jnp.add, steps=(4, 2, 1))
            ent = _allreduce(ent, lanes, jnp.add, steps=(4, 2, 1))
            gsel = (lanes >> 1) == g
            stage_lp = jnp.where(gsel, _lane_perm(lp, row_pick), stage_lp)
            stage_ent = jnp.where(gsel, _lane_perm(ent, row_pick), stage_ent)

    st_lp[...] = stage_lp
    st_ent[...] = stage_ent
    pltpu.sync_copy(st_lp, out_hbm.at[0, wid])
    pltpu.sync_copy(st_ent, out_hbm.at[1, wid])


def _sc_main(policy_output, actions_flat):
    mesh = plsc.VectorSubcoreMesh(core_axis_name="c", subcore_axis_name="s")
    k = functools.partial(
        pl.kernel,
        mesh=mesh,
        out_type=jax.ShapeDtypeStruct((2, NW, LANE), jnp.float32),
        scratch_types=[
            pltpu.VMEM((H * V,), jnp.float32),
            pltpu.VMEM((H * V,), jnp.float32),
            pltpu.VMEM((max(LANE, ACT_N),), jnp.int32),
            pltpu.VMEM((LANE,), jnp.float32),
            pltpu.VMEM((LANE,), jnp.float32),
            pltpu.SemaphoreType.DMA,
            pltpu.SemaphoreType.DMA,
        ],
    )(_sc_kernel)
    return k(policy_output, actions_flat)


R_TC = B - R_SC
TBR = 16
NTB = R_TC // TBR


def _tc_body(x_ref, a_ref, out_ref):
    j = pl.program_id(1)
    x = x_ref[...]
    a = a_ref[0, 0, 0, :TBR][:, None]
    m = jnp.max(x, axis=1, keepdims=True)
    d = x - m
    e = jnp.exp(d)
    s = jnp.sum(e, axis=1)
    u = jnp.sum(d * e, axis=1)
    cols = lax.broadcasted_iota(jnp.int32, (TBR, V), 1)
    xa = jnp.sum(jnp.where(cols == a, x, 0.0), axis=1)
    ln_s = jnp.log(s)
    lp = xa - m[:, 0] - ln_s
    ent = ln_s - u / s

    @pl.when(j == 0)
    def _():
        out_ref[...] = jnp.zeros_like(out_ref)

    out_ref[0, 0, :] += lp
    out_ref[0, 1, :] += ent


def _tc_rows(policy_output, actions_t4):
    return pl.pallas_call(
        _tc_body,
        grid=(NTB, H),
        in_specs=[
            pl.BlockSpec((TBR, V), lambda i, j: (R_SC // TBR + i, j)),
            pl.BlockSpec((1, 1, 1, 128), lambda i, j: (j, i, 0, 0)),
        ],
        out_specs=pl.BlockSpec((1, 2, TBR), lambda i, j: (i, 0, 0)),
        out_shape=jax.ShapeDtypeStruct((NTB, 2, TBR), jnp.float32),
    )(policy_output, actions_t4)


def kernel(policy_output, actions):
    sc = _sc_main(policy_output, actions[:R_SC].reshape(-1))
    a_tc = actions.T[:, R_SC:].reshape(H, NTB, 1, TBR)
    a_tc = jnp.pad(a_tc, ((0, 0), (0, 0), (0, 0), (0, 128 - TBR)))
    tc = _tc_rows(policy_output, a_tc)
    tc = tc.transpose(1, 0, 2).reshape(2, R_TC)
    return jnp.concatenate([sc[:, :, :RPW].reshape(2, R_SC), tc], axis=1)


# speedup vs baseline: 1.2176x; 1.2176x over previous
import functools

import jax
import jax.numpy as jnp
from jax import lax
from jax.experimental import pallas as pl
from jax.experimental.pallas import tpu as pltpu
from jax.experimental.pallas import tpu_sc as plsc

B = 128
H = 8
V = 4096
NW = 32
R_SC = 64
RPW = R_SC // NW
LANE = 16
ACT_N = RPW * H
U = 8
LN2 = 0.6931471805599453


def _lane_perm(v, idx):
    return lax.gather(
        v, idx[:, None],
        dimension_numbers=lax.GatherDimensionNumbers(
            offset_dims=(), collapsed_slice_dims=(0,), start_index_map=(0,)),
        slice_sizes=(1,),
        mode=lax.GatherScatterMode.PROMISE_IN_BOUNDS)


def _tree(vs, op):
    vs = list(vs)
    while len(vs) > 1:
        vs = [op(vs[i], vs[i + 1]) for i in range(0, len(vs) - 1, 2)] + (
            [vs[-1]] if len(vs) % 2 else [])
    return vs[0]


def _allreduce(v, lanes, op, steps=(8, 4, 2, 1)):
    for k in steps:
        v = op(v, _lane_perm(v, lanes ^ k))
    return v


def _ln(s, lanes):
    bits = lax.bitcast_convert_type(s, jnp.int32)
    e = (bits >> 23) - 127
    f = lax.bitcast_convert_type((bits & 0x7FFFFF) | 0x3F800000,
                                 jnp.float32)
    t = f - 1.0
    z = t / (t + 2.0)
    z2 = z * z
    y = e.astype(jnp.float32) * LN2 + z * (2.0 + z2 * (2.0 / 3.0 + z2 * 0.4))
    for _ in range(2):
        y = y + (s * jnp.exp(-y) - 1.0)
    return y


FU = 4
NJ = V // (FU * LANE)


def _row_stats(buf, av_g, parity, lanes, carry):
    m_vec, s_vec, u_vec, xa_vec = carry
    base_lane = parity * H
    zeros = jnp.zeros((LANE,), jnp.float32)
    zerosU = tuple(zeros for _ in range(FU))
    neginfU = tuple(jnp.full((LANE,), -jnp.inf, jnp.float32)
                    for _ in range(FU))

    def at_of(h):
        return _lane_perm(av_g, jnp.full((LANE,), base_lane + h, jnp.int32))

    at0 = at_of(0)

    def p1_body(jj, st):
        maccs, xacc, cols = st
        base = jj * (FU * LANE)
        maccs = list(maccs)
        for k in range(FU):
            v = buf[pl.ds(base + k * LANE, LANE)]
            xacc = jnp.where(cols + k * LANE == at0, v, xacc)
            maccs[k] = jnp.maximum(maccs[k], v)
        return tuple(maccs), xacc, cols + FU * LANE

    maccs, xacc, _ = lax.fori_loop(0, NJ, p1_body, (neginfU, zeros, lanes))
    mt = _allreduce(_tree(maccs, jnp.maximum), lanes, jnp.maximum)
    xat = _lane_perm(xacc, at0 & (LANE - 1))

    def head_body(h, c):
        m_vec, s_vec, u_vec, xa_vec, mt, xat = c
        at = at_of(h)

        def fused_body(jj, st):
            maccs, xacc, cols, ss, uu = st
            b1 = h * V + jj * (FU * LANE)
            b2 = b1 - V
            maccs, ss, uu = list(maccs), list(ss), list(uu)
            for k in range(FU):
                v2 = buf[pl.ds(b2 + k * LANE, LANE)]
                d = v2 - mt
                ex = jnp.exp(d)
                ss[k] = ss[k] + ex
                uu[k] = uu[k] + d * ex
                v1 = buf[pl.ds(b1 + k * LANE, LANE)]
                xacc = jnp.where(cols + k * LANE == at, v1, xacc)
                maccs[k] = jnp.maximum(maccs[k], v1)
            return tuple(maccs), xacc, cols + FU * LANE, tuple(ss), tuple(uu)

        maccs, xacc, _, ss, uu = lax.fori_loop(
            0, NJ, fused_body, (neginfU, zeros, lanes, zerosU, zerosU))
        st = _allreduce(_tree(ss, jnp.add), lanes, jnp.add)
        ut = _allreduce(_tree(uu, jnp.add), lanes, jnp.add)
        sel = lanes == (base_lane + h - 1)
        c_out = (jnp.where(sel, mt, m_vec),
                 jnp.where(sel, st, s_vec),
                 jnp.where(sel, ut, u_vec),
                 jnp.where(sel, xat, xa_vec),
                 _allreduce(_tree(maccs, jnp.maximum), lanes, jnp.maximum),
                 _lane_perm(xacc, at & (LANE - 1)))
        return c_out

    m_vec, s_vec, u_vec, xa_vec, mt, xat = lax.fori_loop(
        1, H, head_body, (m_vec, s_vec, u_vec, xa_vec, mt, xat))

    def p2_body(jj, st):
        ss, uu = st
        base = (H - 1) * V + jj * (FU * LANE)
        ss, uu = list(ss), list(uu)
        for k in range(FU):
            v = buf[pl.ds(base + k * LANE, LANE)]
            d = v - mt
            ex = jnp.exp(d)
            ss[k] = ss[k] + ex
            uu[k] = uu[k] + d * ex
        return tuple(ss), tuple(uu)

    ss, uu = lax.fori_loop(0, NJ, p2_body, (zerosU, zerosU))
    st = _allreduce(_tree(ss, jnp.add), lanes, jnp.add)
    ut = _allreduce(_tree(uu, jnp.add), lanes, jnp.add)
    sel = lanes == (base_lane + H - 1)
    return (jnp.where(sel, mt, m_vec),
            jnp.where(sel, st, s_vec),
            jnp.where(sel, ut, u_vec),
            jnp.where(sel, xat, xa_vec))


def _sc_kernel(policy_hbm, actions_hbm, out_hbm, buf0, buf1, act_v,
               st_lp, st_ent, sem0, sem1):
    wid = lax.axis_index("s") * 2 + lax.axis_index("c")
    r0 = wid * RPW
    lanes = lax.iota(jnp.int32, LANE)

    pltpu.sync_copy(actions_hbm.at[pl.ds(wid * ACT_N, ACT_N)],
                    act_v.at[pl.ds(0, ACT_N)])

    bufs = (buf0, buf1)
    sems = (sem0, sem1)
    copies = [None] * RPW
    for i in range(min(2, RPW)):
        copies[i] = pltpu.async_copy(
            policy_hbm.at[r0 + i], bufs[i % 2], sems[i % 2])

    stage_lp = jnp.zeros((LANE,), jnp.float32)
    stage_ent = jnp.zeros((LANE,), jnp.float32)
    row_pick = (lanes & 1) * 8
    zeros = jnp.zeros((LANE,), jnp.float32)

    carry = (zeros, zeros, zeros, zeros)
    for i in range(RPW):
        g, parity = i // 2, i % 2
        if parity == 0:
            carry = (zeros, zeros, zeros, zeros)
        av_g = act_v[pl.ds(min(g * LANE, max(0, ACT_N - LANE)), LANE)]
        copies[i].wait()
        carry = _row_stats(bufs[i % 2], av_g, parity, lanes, carry)
        if i + 2 < RPW:
            copies[i + 2] = pltpu.async_copy(
                policy_hbm.at[r0 + i + 2], bufs[i % 2], sems[i % 2])
        if parity == 1 or i == RPW - 1:
            m_vec, s_vec, u_vec, xa_vec = carry
            ln_s = _ln(s_vec, lanes)
            lp = xa_vec - m_vec - ln_s
            ent = ln_s - u_vec / s_vec
            lp = _allreduce(lp, lanes, jnp.add, steps=(4, 2, 1))
            ent = _allreduce(ent, lanes, jnp.add, steps=(4, 2, 1))
            gsel = (lanes >> 1) == g
            stage_lp = jnp.where(gsel, _lane_perm(lp, row_pick), stage_lp)
            stage_ent = jnp.where(gsel, _lane_perm(ent, row_pick), stage_ent)

    st_lp[...] = stage_lp
    st_ent[...] = stage_ent
    pltpu.sync_copy(st_lp, out_hbm.at[0, wid])
    pltpu.sync_copy(st_ent, out_hbm.at[1, wid])


def _sc_main(policy_output, actions_flat):
    mesh = plsc.VectorSubcoreMesh(core_axis_name="c", subcore_axis_name="s")
    k = functools.partial(
        pl.kernel,
        mesh=mesh,
        out_type=jax.ShapeDtypeStruct((2, NW, LANE), jnp.float32),
        scratch_types=[
            pltpu.VMEM((H * V,), jnp.float32),
            pltpu.VMEM((H * V,), jnp.float32),
            pltpu.VMEM((max(LANE, ACT_N),), jnp.int32),
            pltpu.VMEM((LANE,), jnp.float32),
            pltpu.VMEM((LANE,), jnp.float32),
            pltpu.SemaphoreType.DMA,
            pltpu.SemaphoreType.DMA,
        ],
    )(_sc_kernel)
    return k(policy_output, actions_flat)


R_TC = B - R_SC
TBR = 32
NTB = R_TC // TBR


def _tc_body(x_ref, a_ref, out_ref):
    j = pl.program_id(1)
    x = x_ref[...]
    a = a_ref[0, 0, 0, :TBR][:, None]
    m = jnp.max(x, axis=1, keepdims=True)
    d = x - m
    e = jnp.exp(d)
    s = jnp.sum(e, axis=1)
    u = jnp.sum(d * e, axis=1)
    cols = lax.broadcasted_iota(jnp.int32, (TBR, V), 1)
    xa = jnp.sum(jnp.where(cols == a, x, 0.0), axis=1)
    ln_s = jnp.log(s)
    lp = xa - m[:, 0] - ln_s
    ent = ln_s - u / s

    @pl.when(j == 0)
    def _():
        out_ref[...] = jnp.zeros_like(out_ref)

    out_ref[0, 0, :] += lp
    out_ref[0, 1, :] += ent


def _tc_rows(policy_output, actions_t4):
    return pl.pallas_call(
        _tc_body,
        grid=(NTB, H),
        in_specs=[
            pl.BlockSpec((TBR, V), lambda i, j: (R_SC // TBR + i, j)),
            pl.BlockSpec((1, 1, 1, 128), lambda i, j: (j, i, 0, 0)),
        ],
        out_specs=pl.BlockSpec((1, 2, TBR), lambda i, j: (i, 0, 0)),
        out_shape=jax.ShapeDtypeStruct((NTB, 2, TBR), jnp.float32),
    )(policy_output, actions_t4)


def kernel(policy_output, actions):
    sc = _sc_main(policy_output, actions[:R_SC].reshape(-1))
    a_tc = actions.T[:, R_SC:].reshape(H, NTB, 1, TBR)
    a_tc = jnp.pad(a_tc, ((0, 0), (0, 0), (0, 0), (0, 128 - TBR)))
    tc = _tc_rows(policy_output, a_tc)
    tc = tc.transpose(1, 0, 2).reshape(2, R_TC)
    return jnp.concatenate([sc[:, :, :RPW].reshape(2, R_SC), tc], axis=1)


# speedup vs baseline: 1.2259x; 1.0069x over previous
import functools

import jax
import jax.numpy as jnp
from jax import lax
from jax.experimental import pallas as pl
from jax.experimental.pallas import tpu as pltpu
from jax.experimental.pallas import tpu_sc as plsc

B = 128
H = 8
V = 4096
NW = 32
R_SC = 64
RPW = R_SC // NW
LANE = 16
ACT_N = RPW * H
LN2 = 0.6931471805599453


def _lane_perm(v, idx):
    return lax.gather(
        v, idx[:, None],
        dimension_numbers=lax.GatherDimensionNumbers(
            offset_dims=(), collapsed_slice_dims=(0,), start_index_map=(0,)),
        slice_sizes=(1,),
        mode=lax.GatherScatterMode.PROMISE_IN_BOUNDS)


def _tree(vs, op):
    vs = list(vs)
    while len(vs) > 1:
        vs = [op(vs[i], vs[i + 1]) for i in range(0, len(vs) - 1, 2)] + (
            [vs[-1]] if len(vs) % 2 else [])
    return vs[0]


def _allreduce(v, lanes, op, steps=(8, 4, 2, 1)):
    for k in steps:
        v = op(v, _lane_perm(v, lanes ^ k))
    return v


def _ln(s, lanes):
    bits = lax.bitcast_convert_type(s, jnp.int32)
    e = (bits >> 23) - 127
    f = lax.bitcast_convert_type((bits & 0x7FFFFF) | 0x3F800000,
                                 jnp.float32)
    t = f - 1.0
    z = t / (t + 2.0)
    z2 = z * z
    y = e.astype(jnp.float32) * LN2 + z * (2.0 + z2 * (2.0 / 3.0 + z2 * 0.4))
    for _ in range(2):
        y = y + (s * jnp.exp(-y) - 1.0)
    return y


FU = 4
NJ = V // (FU * LANE)


def _row_stats(buf, av_g, parity, lanes, carry):
    m_vec, s_vec, u_vec, xa_vec = carry
    base_lane = parity * H
    zeros = jnp.zeros((LANE,), jnp.float32)
    zerosU = tuple(zeros for _ in range(FU))
    neginfU = tuple(jnp.full((LANE,), -jnp.inf, jnp.float32)
                    for _ in range(FU))

    def at_of(h):
        return _lane_perm(av_g, jnp.full((LANE,), base_lane + h, jnp.int32))

    at0 = at_of(0)

    def p1_body(jj, st):
        maccs, xacc, cols = st
        base = jj * (FU * LANE)
        maccs = list(maccs)
        for k in range(FU):
            v = buf[pl.ds(base + k * LANE, LANE)]
            xacc = jnp.where(cols + k * LANE == at0, v, xacc)
            maccs[k] = jnp.maximum(maccs[k], v)
        return tuple(maccs), xacc, cols + FU * LANE

    maccs, xacc, _ = lax.fori_loop(0, NJ, p1_body, (neginfU, zeros, lanes))
    mt = _allreduce(_tree(maccs, jnp.maximum), lanes, jnp.maximum)
    xat = _lane_perm(xacc, at0 & (LANE - 1))

    def head_body(h, c):
        m_vec, s_vec, u_vec, xa_vec, mt, xat = c
        at = at_of(h)

        def fused_body(jj, st):
            maccs, xacc, cols, ss, uu = st
            b1 = h * V + jj * (FU * LANE)
            b2 = b1 - V
            maccs, ss, uu = list(maccs), list(ss), list(uu)
            for k in range(FU):
                v2 = buf[pl.ds(b2 + k * LANE, LANE)]
                d = v2 - mt
                ex = jnp.exp(d)
                ss[k] = ss[k] + ex
                uu[k] = uu[k] + d * ex
                v1 = buf[pl.ds(b1 + k * LANE, LANE)]
                xacc = jnp.where(cols + k * LANE == at, v1, xacc)
                maccs[k] = jnp.maximum(maccs[k], v1)
            return tuple(maccs), xacc, cols + FU * LANE, tuple(ss), tuple(uu)

        maccs, xacc, _, ss, uu = lax.fori_loop(
            0, NJ, fused_body, (neginfU, zeros, lanes, zerosU, zerosU))
        st = _allreduce(_tree(ss, jnp.add), lanes, jnp.add)
        ut = _allreduce(_tree(uu, jnp.add), lanes, jnp.add)
        sel = lanes == (base_lane + h - 1)
        c_out = (jnp.where(sel, mt, m_vec),
                 jnp.where(sel, st, s_vec),
                 jnp.where(sel, ut, u_vec),
                 jnp.where(sel, xat, xa_vec),
                 _allreduce(_tree(maccs, jnp.maximum), lanes, jnp.maximum),
                 _lane_perm(xacc, at & (LANE - 1)))
        return c_out

    m_vec, s_vec, u_vec, xa_vec, mt, xat = lax.fori_loop(
        1, H, head_body, (m_vec, s_vec, u_vec, xa_vec, mt, xat))

    def p2_body(jj, st):
        ss, uu = st
        base = (H - 1) * V + jj * (FU * LANE)
        ss, uu = list(ss), list(uu)
        for k in range(FU):
            v = buf[pl.ds(base + k * LANE, LANE)]
            d = v - mt
            ex = jnp.exp(d)
            ss[k] = ss[k] + ex
            uu[k] = uu[k] + d * ex
        return tuple(ss), tuple(uu)

    ss, uu = lax.fori_loop(0, NJ, p2_body, (zerosU, zerosU))
    st = _allreduce(_tree(ss, jnp.add), lanes, jnp.add)
    ut = _allreduce(_tree(uu, jnp.add), lanes, jnp.add)
    sel = lanes == (base_lane + H - 1)
    return (jnp.where(sel, mt, m_vec),
            jnp.where(sel, st, s_vec),
            jnp.where(sel, ut, u_vec),
            jnp.where(sel, xat, xa_vec))


def _sc_kernel(policy_hbm, actions_hbm, out_hbm, buf0, buf1, act_v,
               st_lp, st_ent, sem0, sem1):
    wid = lax.axis_index("s") * 2 + lax.axis_index("c")
    r0 = wid * RPW
    lanes = lax.iota(jnp.int32, LANE)

    pltpu.sync_copy(actions_hbm.at[pl.ds(wid * ACT_N, ACT_N)],
                    act_v.at[pl.ds(0, ACT_N)])

    bufs = (buf0, buf1)
    sems = (sem0, sem1)
    copies = [None] * RPW
    for i in range(min(2, RPW)):
        copies[i] = pltpu.async_copy(
            policy_hbm.at[r0 + i], bufs[i % 2], sems[i % 2])

    stage_lp = jnp.zeros((LANE,), jnp.float32)
    stage_ent = jnp.zeros((LANE,), jnp.float32)
    row_pick = (lanes & 1) * 8
    zeros = jnp.zeros((LANE,), jnp.float32)

    carry = (zeros, zeros, zeros, zeros)
    for i in range(RPW):
        g, parity = i // 2, i % 2
        if parity == 0:
            carry = (zeros, zeros, zeros, zeros)
        av_g = act_v[pl.ds(min(g * LANE, max(0, ACT_N - LANE)), LANE)]
        copies[i].wait()
        carry = _row_stats(bufs[i % 2], av_g, parity, lanes, carry)
        if i + 2 < RPW:
            copies[i + 2] = pltpu.async_copy(
                policy_hbm.at[r0 + i + 2], bufs[i % 2], sems[i % 2])
        if parity == 1 or i == RPW - 1:
            m_vec, s_vec, u_vec, xa_vec = carry
            ln_s = _ln(s_vec, lanes)
            lp = xa_vec - m_vec - ln_s
            ent = ln_s - u_vec / s_vec
            lp = _allreduce(lp, lanes, jnp.add, steps=(4, 2, 1))
            ent = _allreduce(ent, lanes, jnp.add, steps=(4, 2, 1))
            gsel = (lanes >> 1) == g
            stage_lp = jnp.where(gsel, _lane_perm(lp, row_pick), stage_lp)
            stage_ent = jnp.where(gsel, _lane_perm(ent, row_pick), stage_ent)

    st_lp[...] = stage_lp
    st_ent[...] = stage_ent
    pltpu.sync_copy(st_lp, out_hbm.at[0, wid])
    pltpu.sync_copy(st_ent, out_hbm.at[1, wid])


def _sc_main(policy_output, actions_flat):
    mesh = plsc.VectorSubcoreMesh(core_axis_name="c", subcore_axis_name="s")
    k = functools.partial(
        pl.kernel,
        mesh=mesh,
        out_type=jax.ShapeDtypeStruct((2, NW, LANE), jnp.float32),
        scratch_types=[
            pltpu.VMEM((H * V,), jnp.float32),
            pltpu.VMEM((H * V,), jnp.float32),
            pltpu.VMEM((max(LANE, ACT_N),), jnp.int32),
            pltpu.VMEM((LANE,), jnp.float32),
            pltpu.VMEM((LANE,), jnp.float32),
            pltpu.SemaphoreType.DMA,
            pltpu.SemaphoreType.DMA,
        ],
    )(_sc_kernel)
    return k(policy_output, actions_flat)


R_TC = B - R_SC
TBR = 32
NTB = R_TC // TBR


def _tc_body(x_ref, a_ref, out_ref):
    j = pl.program_id(1)
    x = x_ref[...]
    a = a_ref[0, 0, 0, :TBR][:, None]
    m = jnp.max(x, axis=1, keepdims=True)
    d = x - m
    e = jnp.exp(d)
    s = jnp.sum(e, axis=1)
    u = jnp.sum(d * e, axis=1)
    cols = lax.broadcasted_iota(jnp.int32, (TBR, V), 1)
    xa = jnp.sum(jnp.where(cols == a, x, 0.0), axis=1)
    ln_s = jnp.log(s)
    lp = xa - m[:, 0] - ln_s
    ent = ln_s - u / s

    @pl.when(j == 0)
    def _():
        out_ref[...] = jnp.zeros_like(out_ref)

    out_ref[0, 0, :] += lp
    out_ref[0, 1, :] += ent


def _tc_rows(policy_output, actions_t4):
    return pl.pallas_call(
        _tc_body,
        grid=(NTB, H),
        in_specs=[
            pl.BlockSpec((TBR, V), lambda i, j: (R_SC // TBR + i, j)),
            pl.BlockSpec((1, 1, 1, 128), lambda i, j: (j, i, 0, 0)),
        ],
        out_specs=pl.BlockSpec((1, 2, TBR), lambda i, j: (i, 0, 0)),
        out_shape=jax.ShapeDtypeStruct((NTB, 2, TBR), jnp.float32),
    )(policy_output, actions_t4)


def kernel(policy_output, actions):
    sc = _sc_main(policy_output, actions[:R_SC].reshape(-1))
    a_tc = actions.T[:, R_SC:].reshape(H, NTB, 1, TBR)
    a_tc = jnp.pad(a_tc, ((0, 0), (0, 0), (0, 0), (0, 128 - TBR)))
    tc = _tc_rows(policy_output, a_tc)
    tc = tc.transpose(1, 0, 2).reshape(2, R_TC)
    return jnp.concatenate([sc[:, :, :RPW].reshape(2, R_SC), tc], axis=1)
